# Initial kernel scaffold; baseline (speedup 1.0000x reference)
import jax
import jax.numpy as jnp
from jax.experimental import pallas as pl


def kernel(x, conv_w, conv_b, cls_w):
    raise NotImplementedError("write your pallas kernel here")


# speedup vs baseline: 2.1622x; 2.1622x over previous
import functools

import jax
import jax.numpy as jnp
from jax import lax
from jax.experimental import pallas as pl
from jax.experimental.pallas import tpu as pltpu
from jax.experimental.pallas import tpu_sc as plsc

B, T, F = 16, 512, 2048
C_OUT = 2048
NUM_CLASSES = 20
K = 64

NC = 4
CT = C_OUT // NC

_HI = jax.lax.Precision.DEFAULT


def _dot(a, b):
    return lax.dot_general(a, b, (((1,), (0,)), ((), ())),
                           precision=_HI, preferred_element_type=jnp.float32)


def _conv_body(x_ref, w_ref, b_ref, wc_ref, f_ref, mag_ref, ct_ref, sm_ref):
    c = pl.program_id(0)
    b = pl.program_id(1)
    xb = x_ref[0]
    p0 = _dot(xb, w_ref[0])
    p1 = _dot(xb, w_ref[1])
    p2 = _dot(xb, w_ref[2])
    z = jnp.zeros((1, CT), jnp.float32)
    acc = (p1
           + jnp.concatenate([z, p0[:-1]], axis=0)
           + jnp.concatenate([p2[1:], z], axis=0))
    feat = jnp.maximum(acc + b_ref[0:1, :], 0.0)
    f_ref[0] = feat

    m2 = jnp.sum(feat * feat, axis=1)[None]
    wc = wc_ref[...]
    ct_part = lax.dot_general(wc, feat, (((0,), (1,)), ((), ())),
                              precision=_HI,
                              preferred_element_type=jnp.float32)
    tm_part = _dot(feat, wc)

    row = pl.ds(b, 1)

    @pl.when(c == 0)
    def _init():
        mag_ref[row, :] = m2
        ct_ref[row] = ct_part[None]
        sm_ref[row] = tm_part[None]

    @pl.when(c > 0)
    def _acc():
        mag_ref[row, :] = mag_ref[row, :] + m2
        ct_ref[row] = ct_ref[row] + ct_part[None]
        sm_ref[row] = sm_ref[row] + tm_part[None]

    @pl.when(c == NC - 1)
    def _final():
        mag_ref[row, :] = jnp.sqrt(mag_ref[row, :])
        zt = sm_ref[row]
        m = jnp.max(zt, axis=2, keepdims=True)
        e = jnp.exp(zt - m)
        sm_ref[row] = e / jnp.sum(e, axis=2, keepdims=True)


def _select_body(mag_ref, ct_ref, idx_ref, sa_ref, sb_ref):
    mag = mag_ref[...]
    iota = lax.broadcasted_iota(jnp.int32, (B, T), 1)
    big = jnp.int32(1 << 30)
    neg = jnp.float32(-jnp.inf)

    def extract(vals):
        cur = vals
        idxs = []
        for _ in range(K):
            m = jnp.max(cur, axis=1, keepdims=True)
            first = jnp.min(jnp.where(cur == m, iota, big),
                            axis=1, keepdims=True)
            idxs.append(first)
            cur = jnp.where(iota == first, neg, cur)
        return jnp.concatenate(idxs, axis=1), (cur == neg)

    act_idx, _ = extract(mag)
    rev = jnp.max(mag, axis=1, keepdims=True) - mag
    bkg_idx, bkg_mask = extract(rev)

    ct = ct_ref[...]
    maskf = bkg_mask.astype(jnp.float32)[:, None, :]
    sbkg = jnp.sum(ct * maskf, axis=2) / jnp.float32(K)

    bits = lax.bitcast_convert_type(ct, jnp.int32)
    flip = jnp.int32(0x7FFFFFFF)
    keys = jnp.where(bits >= 0, bits, bits ^ flip)
    lo = jnp.full((B, NUM_CLASSES, 1), jnp.iinfo(jnp.int32).min, jnp.int32)
    hi = jnp.full((B, NUM_CLASSES, 1), jnp.iinfo(jnp.int32).max, jnp.int32)
    for _ in range(32):
        mid = (lo >> 1) + (hi >> 1) + (lo & hi & 1)
        cnt = jnp.sum((keys > mid).astype(jnp.int32), axis=2, keepdims=True)
        pred = cnt <= K - 1
        hi = jnp.where(pred, mid, hi)
        lo = jnp.where(pred, lo, mid + 1)
    tau_bits = jnp.where(lo >= 0, lo, lo ^ flip)
    tau = lax.bitcast_convert_type(tau_bits, jnp.float32)
    gt = keys > lo
    cnt_gt = jnp.sum(gt.astype(jnp.float32), axis=2, keepdims=True)
    sum_gt = jnp.sum(jnp.where(gt, ct, 0.0), axis=2, keepdims=True)
    sact = ((sum_gt + (jnp.float32(K) - cnt_gt) * tau) / jnp.float32(K))[:, :, 0]

    def smax(v):
        m = jnp.max(v, axis=1, keepdims=True)
        e = jnp.exp(v - m)
        return e / jnp.sum(e, axis=1, keepdims=True)

    sa_ref[...] = smax(sact)
    sb_ref[...] = smax(sbkg)
    boff = lax.broadcasted_iota(jnp.int32, (B, 1), 0) * T
    idx_ref[...] = jnp.concatenate([act_idx, bkg_idx], axis=1) + boff


_NW = 32
_RPW = (2 * B * K) // _NW
_CH = 16


@functools.cache
def _make_sc_gather():
    mesh = plsc.VectorSubcoreMesh(core_axis_name="c", subcore_axis_name="s")

    @functools.partial(
        pl.kernel, mesh=mesh,
        out_type=jax.ShapeDtypeStruct((2 * B * K, C_OUT), jnp.float32),
        scratch_types=[
            pltpu.VMEM((_CH,), jnp.int32),
            pltpu.VMEM((_CH, C_OUT), jnp.float32),
            pltpu.SemaphoreType.DMA,
        ],
    )
    def sc_gather(table_hbm, idx_hbm, out_hbm, idx_v, rows_v, sem):
        wid = lax.axis_index("s") * 2 + lax.axis_index("c")
        base = wid * _RPW
        for j in range(_RPW // _CH):
            r0 = base + j * _CH
            pltpu.sync_copy(idx_hbm.at[pl.ds(r0, _CH)], idx_v)
            pltpu.async_copy(table_hbm.at[idx_v], rows_v, sem).wait()
            pltpu.sync_copy(rows_v, out_hbm.at[pl.ds(r0, _CH)])

    return sc_gather


def kernel(x, conv_w, conv_b, cls_w):
    wt = jnp.transpose(conv_w, (2, 1, 0))
    wcls = jnp.transpose(cls_w[:, :, 0], (1, 0))
    bias2 = jnp.broadcast_to(conv_b[None, :], (8, C_OUT))

    features, mag, cas_t, cas_sm = pl.pallas_call(
        _conv_body,
        grid=(NC, B),
        in_specs=[
            pl.BlockSpec((1, T, F), lambda c, b: (b, 0, 0)),
            pl.BlockSpec((3, F, CT), lambda c, b: (0, 0, c)),
            pl.BlockSpec((8, CT), lambda c, b: (0, c)),
            pl.BlockSpec((CT, NUM_CLASSES), lambda c, b: (c, 0)),
        ],
        out_specs=[
            pl.BlockSpec((1, T, CT), lambda c, b: (b, 0, c)),
            pl.BlockSpec((B, T), lambda c, b: (0, 0)),
            pl.BlockSpec((B, NUM_CLASSES, T), lambda c, b: (0, 0, 0)),
            pl.BlockSpec((B, T, NUM_CLASSES), lambda c, b: (0, 0, 0)),
        ],
        out_shape=[
            jax.ShapeDtypeStruct((B, T, C_OUT), jnp.float32),
            jax.ShapeDtypeStruct((B, T), jnp.float32),
            jax.ShapeDtypeStruct((B, NUM_CLASSES, T), jnp.float32),
            jax.ShapeDtypeStruct((B, T, NUM_CLASSES), jnp.float32),
        ],
        compiler_params=pltpu.CompilerParams(
            dimension_semantics=("arbitrary", "arbitrary"),
        ),
    )(x, wt, bias2, wcls)

    idxg, score_act, score_bkg = pl.pallas_call(
        _select_body,
        out_shape=[
            jax.ShapeDtypeStruct((B, 2 * K), jnp.int32),
            jax.ShapeDtypeStruct((B, NUM_CLASSES), jnp.float32),
            jax.ShapeDtypeStruct((B, NUM_CLASSES), jnp.float32),
        ],
    )(mag, cas_t)

    g = _make_sc_gather()(features.reshape(B * T, C_OUT), idxg.reshape(2 * B * K))
    g = g.reshape(B, 2 * K, C_OUT)
    feat_act = g[:, :K]
    feat_bkg = g[:, K:]
    return score_act, score_bkg, feat_act, feat_bkg, features, cas_sm


# speedup vs baseline: 2.1956x; 1.0154x over previous
import functools

import jax
import jax.numpy as jnp
from jax import lax
from jax.experimental import pallas as pl
from jax.experimental.pallas import tpu as pltpu
from jax.experimental.pallas import tpu_sc as plsc

B, T, F = 16, 512, 2048
C_OUT = 2048
NUM_CLASSES = 20
K = 64

NC = 2
CT = C_OUT // NC

_HI = jax.lax.Precision.DEFAULT


def _dot(a, b):
    return lax.dot_general(a, b, (((1,), (0,)), ((), ())),
                           precision=_HI, preferred_element_type=jnp.float32)


def _conv_body(x_ref, w_ref, b_ref, wc_ref, f_ref, mag_ref, ct_ref, sm_ref):
    c = pl.program_id(0)
    b = pl.program_id(1)
    xb = x_ref[0]
    p0 = _dot(xb, w_ref[0])
    p1 = _dot(xb, w_ref[1])
    p2 = _dot(xb, w_ref[2])
    z = jnp.zeros((1, CT), jnp.float32)
    acc = (p1
           + jnp.concatenate([z, p0[:-1]], axis=0)
           + jnp.concatenate([p2[1:], z], axis=0))
    feat = jnp.maximum(acc + b_ref[0:1, :], 0.0)
    f_ref[0] = feat

    m2 = jnp.sum(feat * feat, axis=1)[None]
    wc = wc_ref[...]
    ct_part = lax.dot_general(wc, feat, (((0,), (1,)), ((), ())),
                              precision=_HI,
                              preferred_element_type=jnp.float32)
    tm_part = _dot(feat, wc)

    row = pl.ds(b, 1)

    @pl.when(c == 0)
    def _init():
        mag_ref[row, :] = m2
        ct_ref[row] = ct_part[None]
        sm_ref[row] = tm_part[None]

    @pl.when(c > 0)
    def _acc():
        mag_ref[row, :] = mag_ref[row, :] + m2
        ct_ref[row] = ct_ref[row] + ct_part[None]
        sm_ref[row] = sm_ref[row] + tm_part[None]

    @pl.when(c == NC - 1)
    def _final():
        mag_ref[row, :] = jnp.sqrt(mag_ref[row, :])
        zt = sm_ref[row]
        m = jnp.max(zt, axis=2, keepdims=True)
        e = jnp.exp(zt - m)
        sm_ref[row] = e / jnp.sum(e, axis=2, keepdims=True)


def _select_body(mag_ref, ct_ref, idx_ref, sa_ref, sb_ref):
    mag = mag_ref[...]
    iota = lax.broadcasted_iota(jnp.int32, (B, T), 1)
    big = jnp.int32(1 << 30)
    neg = jnp.float32(-jnp.inf)

    def extract(vals):
        cur = vals
        idxs = []
        for _ in range(K):
            m = jnp.max(cur, axis=1, keepdims=True)
            first = jnp.min(jnp.where(cur == m, iota, big),
                            axis=1, keepdims=True)
            idxs.append(first)
            cur = jnp.where(iota == first, neg, cur)
        return jnp.concatenate(idxs, axis=1), (cur == neg)

    act_idx, _ = extract(mag)
    rev = jnp.max(mag, axis=1, keepdims=True) - mag
    bkg_idx, bkg_mask = extract(rev)

    ct = ct_ref[...]
    maskf = bkg_mask.astype(jnp.float32)[:, None, :]
    sbkg = jnp.sum(ct * maskf, axis=2) / jnp.float32(K)

    bits = lax.bitcast_convert_type(ct, jnp.int32)
    flip = jnp.int32(0x7FFFFFFF)
    keys = jnp.where(bits >= 0, bits, bits ^ flip)
    lo = jnp.full((B, NUM_CLASSES, 1), jnp.iinfo(jnp.int32).min, jnp.int32)
    hi = jnp.full((B, NUM_CLASSES, 1), jnp.iinfo(jnp.int32).max, jnp.int32)
    for _ in range(32):
        mid = (lo >> 1) + (hi >> 1) + (lo & hi & 1)
        cnt = jnp.sum((keys > mid).astype(jnp.int32), axis=2, keepdims=True)
        pred = cnt <= K - 1
        hi = jnp.where(pred, mid, hi)
        lo = jnp.where(pred, lo, mid + 1)
    tau_bits = jnp.where(lo >= 0, lo, lo ^ flip)
    tau = lax.bitcast_convert_type(tau_bits, jnp.float32)
    gt = keys > lo
    cnt_gt = jnp.sum(gt.astype(jnp.float32), axis=2, keepdims=True)
    sum_gt = jnp.sum(jnp.where(gt, ct, 0.0), axis=2, keepdims=True)
    sact = ((sum_gt + (jnp.float32(K) - cnt_gt) * tau) / jnp.float32(K))[:, :, 0]

    def smax(v):
        m = jnp.max(v, axis=1, keepdims=True)
        e = jnp.exp(v - m)
        return e / jnp.sum(e, axis=1, keepdims=True)

    sa_ref[...] = smax(sact)
    sb_ref[...] = smax(sbkg)
    boff = lax.broadcasted_iota(jnp.int32, (B, 1), 0) * T
    idx_ref[...] = jnp.concatenate([act_idx + boff, bkg_idx + boff], axis=0)


_NW = 32
_RPW = (2 * B * K) // _NW
_CH = 16


@functools.cache
def _make_sc_gather():
    mesh = plsc.VectorSubcoreMesh(core_axis_name="c", subcore_axis_name="s")

    @functools.partial(
        pl.kernel, mesh=mesh,
        out_type=(
            jax.ShapeDtypeStruct((B * K, C_OUT), jnp.float32),
            jax.ShapeDtypeStruct((B * K, C_OUT), jnp.float32),
        ),
        scratch_types=[
            pltpu.VMEM((_CH,), jnp.int32),
            pltpu.VMEM((_CH, C_OUT), jnp.float32),
            pltpu.SemaphoreType.DMA,
        ],
    )
    def sc_gather(table_hbm, idx_hbm, act_hbm, bkg_hbm, idx_v, rows_v, sem):
        wid = lax.axis_index("s") * 2 + lax.axis_index("c")
        base = wid * _RPW

        def run(out_hbm, off):
            for j in range(_RPW // _CH):
                r0 = base + j * _CH
                pltpu.sync_copy(idx_hbm.at[pl.ds(r0, _CH)], idx_v)
                pltpu.async_copy(table_hbm.at[idx_v], rows_v, sem).wait()
                pltpu.sync_copy(rows_v, out_hbm.at[pl.ds(r0 - off, _CH)])

        @pl.when(wid < _NW // 2)
        def _act():
            run(act_hbm, 0)

        @pl.when(wid >= _NW // 2)
        def _bkg():
            run(bkg_hbm, B * K)

    return sc_gather


def kernel(x, conv_w, conv_b, cls_w):
    xb16 = x.astype(jnp.bfloat16)
    wt = jnp.transpose(conv_w, (2, 1, 0)).astype(jnp.bfloat16)
    wcls = jnp.transpose(cls_w[:, :, 0], (1, 0))
    bias2 = jnp.broadcast_to(conv_b[None, :], (8, C_OUT))

    features, mag, cas_t, cas_sm = pl.pallas_call(
        _conv_body,
        grid=(NC, B),
        in_specs=[
            pl.BlockSpec((1, T, F), lambda c, b: (b, 0, 0)),
            pl.BlockSpec((3, F, CT), lambda c, b: (0, 0, c)),
            pl.BlockSpec((8, CT), lambda c, b: (0, c)),
            pl.BlockSpec((CT, NUM_CLASSES), lambda c, b: (c, 0)),
        ],
        out_specs=[
            pl.BlockSpec((1, T, CT), lambda c, b: (b, 0, c)),
            pl.BlockSpec((B, T), lambda c, b: (0, 0)),
            pl.BlockSpec((B, NUM_CLASSES, T), lambda c, b: (0, 0, 0)),
            pl.BlockSpec((B, T, NUM_CLASSES), lambda c, b: (0, 0, 0)),
        ],
        out_shape=[
            jax.ShapeDtypeStruct((B, T, C_OUT), jnp.float32),
            jax.ShapeDtypeStruct((B, T), jnp.float32),
            jax.ShapeDtypeStruct((B, NUM_CLASSES, T), jnp.float32),
            jax.ShapeDtypeStruct((B, T, NUM_CLASSES), jnp.float32),
        ],
        compiler_params=pltpu.CompilerParams(
            dimension_semantics=("arbitrary", "arbitrary"),
            vmem_limit_bytes=128 * 1024 * 1024,
        ),
    )(xb16, wt, bias2, wcls)

    idxg, score_act, score_bkg = pl.pallas_call(
        _select_body,
        out_shape=[
            jax.ShapeDtypeStruct((2 * B, K), jnp.int32),
            jax.ShapeDtypeStruct((B, NUM_CLASSES), jnp.float32),
            jax.ShapeDtypeStruct((B, NUM_CLASSES), jnp.float32),
        ],
    )(mag, cas_t)

    ga, gb = _make_sc_gather()(
        features.reshape(B * T, C_OUT), idxg.reshape(2 * B * K))
    feat_act = ga.reshape(B, K, C_OUT)
    feat_bkg = gb.reshape(B, K, C_OUT)
    return score_act, score_bkg, feat_act, feat_bkg, features, cas_sm
